# baseline (speedup 1.0000x reference)
import functools

import jax
import jax.numpy as jnp
from jax import lax
from jax.experimental import pallas as pl
from jax.experimental.pallas import tpu as pltpu
from jax.experimental.pallas import tpu_sc as plsc

B = 16384
D = 64
NC = 2
NS = 16
NW = NC * NS
BPW = B // NW
CHUNK = 128
NCHUNK = BPW // CHUNK


def _sc_body(users, items, eu_mlp, ei_mlp, eu_mf, ei_mf, umask, imask,
             out_u, out_i, out_mf,
             idx_u, idx_i, bu, bi, bmu, bmi, bfu, bfi, sem):
    wid = lax.axis_index("s") * NC + lax.axis_index("c")
    for ch in range(NCHUNK):
        base = wid * BPW + ch * CHUNK
        pltpu.sync_copy(users.at[pl.ds(base, CHUNK)], idx_u)
        pltpu.sync_copy(items.at[pl.ds(base, CHUNK)], idx_i)
        cps = [
            pltpu.async_copy(eu_mlp.at[idx_u], bu, sem),
            pltpu.async_copy(ei_mlp.at[idx_i], bi, sem),
            pltpu.async_copy(umask.at[idx_u], bmu, sem),
            pltpu.async_copy(imask.at[idx_i], bmi, sem),
            pltpu.async_copy(eu_mf.at[idx_u], bfu, sem),
            pltpu.async_copy(ei_mf.at[idx_i], bfi, sem),
        ]
        for cp in cps:
            cp.wait()

        def row(r, carry):
            for j in range(D // 16):
                sl = pl.ds(j * 16, 16)
                bu[r, sl] = bu[r, sl] * bmu[r, sl].astype(jnp.float32)
                bi[r, sl] = bi[r, sl] * bmi[r, sl].astype(jnp.float32)
                bfu[r, sl] = bfu[r, sl] * bfi[r, sl]
            return carry

        lax.fori_loop(0, CHUNK, row, 0)
        pltpu.sync_copy(bu, out_u.at[pl.ds(base, CHUNK)])
        pltpu.sync_copy(bi, out_i.at[pl.ds(base, CHUNK)])
        pltpu.sync_copy(bfu, out_mf.at[pl.ds(base, CHUNK)])


@functools.cache
def _sc_gather():
    return pl.kernel(
        _sc_body,
        out_type=[jax.ShapeDtypeStruct((B, D), jnp.float32)] * 3,
        mesh=plsc.VectorSubcoreMesh(core_axis_name="c", subcore_axis_name="s"),
        compiler_params=pltpu.CompilerParams(use_tc_tiling_on_sc=False),
        scratch_types=[
            pltpu.VMEM((CHUNK,), jnp.int32),
            pltpu.VMEM((CHUNK,), jnp.int32),
            pltpu.VMEM((CHUNK, D), jnp.float32),
            pltpu.VMEM((CHUNK, D), jnp.float32),
            pltpu.VMEM((CHUNK, D), jnp.int32),
            pltpu.VMEM((CHUNK, D), jnp.int32),
            pltpu.VMEM((CHUNK, D), jnp.float32),
            pltpu.VMEM((CHUNK, D), jnp.float32),
            pltpu.SemaphoreType.DMA,
        ],
    )

BT = 2048


def _tc_body(xu, xi, mf, w1a, w1b, b1, w2a, w2b, b2, out):
    h = jnp.dot(xu[...], w1a[...], preferred_element_type=jnp.float32)
    h = h + jnp.dot(xi[...], w1b[...], preferred_element_type=jnp.float32)
    h = jnp.maximum(h + b1[...], 0.0)
    o = jnp.dot(h, w2a[...], preferred_element_type=jnp.float32)
    o = o + jnp.dot(mf[...], w2b[...], preferred_element_type=jnp.float32)
    out[...] = o + b2[0, 0]


_tc_call = pl.pallas_call(
    _tc_body,
    grid=(B // BT,),
    in_specs=[
        pl.BlockSpec((BT, D), lambda i: (i, 0)),
        pl.BlockSpec((BT, D), lambda i: (i, 0)),
        pl.BlockSpec((BT, D), lambda i: (i, 0)),
        pl.BlockSpec((D, D), lambda i: (0, 0)),
        pl.BlockSpec((D, D), lambda i: (0, 0)),
        pl.BlockSpec((1, D), lambda i: (0, 0)),
        pl.BlockSpec((D, 1), lambda i: (0, 0)),
        pl.BlockSpec((D, 1), lambda i: (0, 0)),
        pl.BlockSpec((1, 1), lambda i: (0, 0)),
    ],
    out_specs=pl.BlockSpec((BT, 1), lambda i: (i, 0)),
    out_shape=jax.ShapeDtypeStruct((B, 1), jnp.float32),
)


def kernel(users, items, emb_user_mlp, emb_item_mlp, emb_user_mf, emb_item_mf,
           user_mask, item_mask, W1, b1, W2, b2):
    xu, xi, mf = _sc_gather()(users, items, emb_user_mlp, emb_item_mlp,
                              emb_user_mf, emb_item_mf, user_mask, item_mask)
    logits = _tc_call(xu, xi, mf,
                      W1[:D], W1[D:], b1.reshape(1, D),
                      W2[:D], W2[D:], b2.reshape(1, 1))
    return logits
